# Initial kernel scaffold; baseline (speedup 1.0000x reference)
import functools

import jax
import jax.numpy as jnp
from jax import lax
from jax.experimental import pallas as pl
from jax.experimental.pallas import tpu as pltpu
from jax.experimental.pallas import tpu_sc as plsc

SEQLEN = 50
BATCH = 4096
D = 64
NTYPES_N = 100
NTYPES_E = 50
NROWS = SEQLEN * BATCH
TOTNNZ = 409600
NBLK = 10
RB = NROWS // NBLK
SEQ_PER_BLK = SEQLEN // NBLK
CHUNK = 256
NCHUNK = RB // CHUNK
WA = 640
WB = 512
NC = 2
NS = 16
NW = NC * NS
PER_W = NROWS // NW
ROWS_PER_TILE = RB // NS


def _prep_body(node_ref, edge_ref, posn_ref, pose_ref, posv_ref,
               fn_ref, fe_ref, pr_ref):
    fn_ref[...] = node_ref[...][None, :, :] + posn_ref[...][:, None, :]
    fe_ref[...] = edge_ref[...][None, :, :] + pose_ref[...][:, None, :]
    pr_ref[...] = jnp.broadcast_to(posv_ref[...][:, None, :],
                                   (SEQLEN, CHUNK, D))


_prep = pl.pallas_call(
    _prep_body,
    out_shape=(
        jax.ShapeDtypeStruct((SEQLEN, NTYPES_N, D), jnp.float32),
        jax.ShapeDtypeStruct((SEQLEN, NTYPES_E, D), jnp.float32),
        jax.ShapeDtypeStruct((SEQLEN, CHUNK, D), jnp.float32),
    ),
)

_vmesh = plsc.VectorSubcoreMesh(core_axis_name="c", subcore_axis_name="s")


@functools.partial(
    pl.kernel,
    out_type=jax.ShapeDtypeStruct((3 * NROWS, D), jnp.float32),
    mesh=_vmesh,
    scratch_types=[
        pltpu.VMEM((WA,), jnp.int32),
        pltpu.VMEM((5, 128), jnp.int32),
        pltpu.VMEM((WA, D), jnp.float32),
        pltpu.VMEM((WB,), jnp.int32),
        pltpu.VMEM((4, 128), jnp.int32),
        pltpu.VMEM((WB,), jnp.float32),
        pltpu.VMEM((16,), jnp.int32),
        pltpu.SMEM((16,), jnp.int32),
        pltpu.SMEM((WB,), jnp.float32),
        pltpu.VMEM_SHARED((RB, D), jnp.float32),
        pltpu.SemaphoreType.DMA,
    ],
)
def _sc_encode(fnode_h, fedge_h, nidx_h, eidx_h, rows_h, cols_h, vals_h,
               vembed_h, posrep_h, bounds_h, out_h,
               idxstage, idx2, gbuf, rowstage, rowsbuf, valsbuf,
               bounds_v, bounds_s, vals_s, acc, sem):
    c = lax.axis_index("c")
    t = lax.axis_index("s")
    wid = t * NC + c
    lane = lax.iota(jnp.int32, 16)

    def phase_a(idx_hbm, table_hbm, stride, out_base):
        @pl.loop(0, PER_W // WA)
        def _(w):
            base = wid * PER_W + w * WA
            pltpu.sync_copy(idx_hbm.at[pl.ds(base, WA)], idxstage)
            for q in range(WA // 128):
                for g8 in range(8):
                    g = q * 8 + g8
                    sl = pl.ds(g * 16, 16)
                    gi = base + g * 16 + lane
                    seqv = lax.shift_right_logical(gi, 12)
                    idx2[q, pl.ds(g8 * 16, 16)] = idxstage[sl] + seqv * stride
            cps = [
                pltpu.async_copy(table_hbm.at[idx2.at[q]],
                                 gbuf.at[pl.ds(q * 128, 128)], sem)
                for q in range(WA // 128)
            ]
            for cp in cps:
                cp.wait()
            pltpu.sync_copy(gbuf, out_h.at[pl.ds(out_base + base, WA)])

    phase_a(nidx_h, fnode_h, NTYPES_N, 0)
    phase_a(eidx_h, fedge_h, NTYPES_E, NROWS)

    pltpu.sync_copy(bounds_h, bounds_v)
    pltpu.sync_copy(bounds_v, bounds_s)

    for bi in range(NBLK // NC):
        b = bi * NC + c
        r0 = b * RB
        sb = bounds_s[b]
        eb = bounds_s[b + 1]

        for ki in range(NCHUNK // NS):
            k = t * (NCHUNK // NS) + ki
            seqq = b * SEQ_PER_BLK + lax.div(k, BATCH // CHUNK)
            pltpu.sync_copy(posrep_h.at[seqq], acc.at[pl.ds(k * CHUNK, CHUNK)])
        plsc.subcore_barrier()

        jlo = lax.div(sb, WB)
        jhi = lax.div(eb + (WB - 1), WB)

        @pl.loop(jlo + t, jhi, step=NS)
        def _(j):
            base = j * WB
            pltpu.sync_copy(cols_h.at[pl.ds(base, WB)],
                            idxstage.at[pl.ds(0, WB)])
            pltpu.sync_copy(rows_h.at[pl.ds(base, WB)], rowstage)
            pltpu.sync_copy(vals_h.at[pl.ds(base, WB)], valsbuf)
            for q in range(WB // 128):
                for g8 in range(8):
                    g = q * 8 + g8
                    sl = pl.ds(g * 16, 16)
                    sl8 = pl.ds(g8 * 16, 16)
                    idx2[q, sl8] = idxstage[sl]
                    rv = rowstage[sl] - r0
                    rv = jnp.minimum(jnp.maximum(rv, 0), RB - 1)
                    rowsbuf[q, sl8] = rv
                    gi = base + g * 16 + lane
                    m = (gi >= sb) & (gi < eb)
                    valsbuf[sl] = jnp.where(m, valsbuf[sl], jnp.float32(0.0))
            cps = [
                pltpu.async_copy(vembed_h.at[idx2.at[q]],
                                 gbuf.at[pl.ds(q * 128, 128)], sem)
                for q in range(WB // 128)
            ]
            for cp in cps:
                cp.wait()
            pltpu.sync_copy(valsbuf, vals_s)

            @pl.loop(0, WB)
            def _(i):
                v = vals_s[i]
                for qq in range(D // 16):
                    slq = pl.ds(qq * 16, 16)
                    gbuf[i, slq] = gbuf[i, slq] * v

            for q in range(WB // 128):
                pltpu.sync_copy(gbuf.at[pl.ds(q * 128, 128)],
                                acc.at[rowsbuf.at[q]], add=True)

        plsc.subcore_barrier()
        pltpu.sync_copy(
            acc.at[pl.ds(t * ROWS_PER_TILE, ROWS_PER_TILE)],
            out_h.at[pl.ds(2 * NROWS + r0 + t * ROWS_PER_TILE,
                           ROWS_PER_TILE)])
        plsc.subcore_barrier()


def kernel(node_idx, edge_idx, val_rows, val_cols, val_vals,
           node_table, edge_table, val_embed, pos_node, pos_edge, pos_val):
    fnode, fedge, posrep = _prep(node_table, edge_table,
                                 pos_node, pos_edge, pos_val)
    fnode = fnode.reshape(SEQLEN * NTYPES_N, D)
    fedge = fedge.reshape(SEQLEN * NTYPES_E, D)
    nidx = node_idx.reshape(-1).astype(jnp.int32)
    eidx = edge_idx.reshape(-1).astype(jnp.int32)
    rows = val_rows.astype(jnp.int32)
    cols = val_cols.astype(jnp.int32)
    edges = jnp.arange(0, NROWS + 1, RB, dtype=jnp.int32)
    bounds = jnp.searchsorted(rows, edges).astype(jnp.int32)
    bounds = jnp.concatenate(
        [bounds, jnp.zeros((16 - NBLK - 1,), jnp.int32)])
    out_flat = _sc_encode(fnode, fedge, nidx, eidx, rows, cols,
                          val_vals, val_embed, posrep, bounds)
    return out_flat.reshape(3 * SEQLEN, BATCH, D)


# speedup vs baseline: 4.0020x; 4.0020x over previous
import functools

import jax
import jax.numpy as jnp
from jax import lax
from jax.experimental import pallas as pl
from jax.experimental.pallas import tpu as pltpu
from jax.experimental.pallas import tpu_sc as plsc

SEQLEN = 50
BATCH = 4096
D = 64
NTYPES_N = 100
NTYPES_E = 50
NROWS = SEQLEN * BATCH
TOTNNZ = 409600
NBLK = 10
RB = NROWS // NBLK
SEQ_PER_BLK = SEQLEN // NBLK
CHUNK = 256
NCHUNK = RB // CHUNK
WA = 640
WB = 512
NC = 2
NS = 16
NW = NC * NS
PER_W = NROWS // NW
ROWS_PER_TILE = RB // NS


def _prep_body(node_ref, edge_ref, posn_ref, pose_ref, posv_ref,
               fn_ref, fe_ref, pr_ref):
    fn_ref[...] = node_ref[...][None, :, :] + posn_ref[...][:, None, :]
    fe_ref[...] = edge_ref[...][None, :, :] + pose_ref[...][:, None, :]
    pr_ref[...] = jnp.broadcast_to(posv_ref[...][:, None, :],
                                   (SEQLEN, CHUNK, D))


_prep = pl.pallas_call(
    _prep_body,
    out_shape=(
        jax.ShapeDtypeStruct((SEQLEN, NTYPES_N, D), jnp.float32),
        jax.ShapeDtypeStruct((SEQLEN, NTYPES_E, D), jnp.float32),
        jax.ShapeDtypeStruct((SEQLEN, CHUNK, D), jnp.float32),
    ),
)

_vmesh = plsc.VectorSubcoreMesh(core_axis_name="c", subcore_axis_name="s")


@functools.partial(
    pl.kernel,
    out_type=jax.ShapeDtypeStruct((3 * NROWS, D), jnp.float32),
    mesh=_vmesh,
    compiler_params=pltpu.CompilerParams(use_tc_tiling_on_sc=False),
    scratch_types=[
        pltpu.VMEM((WA,), jnp.int32),
        pltpu.VMEM((5, 128), jnp.int32),
        pltpu.VMEM((WA, D), jnp.float32),
        pltpu.VMEM((WB,), jnp.int32),
        pltpu.VMEM((4, 128), jnp.int32),
        pltpu.VMEM((32,), jnp.int32),
        pltpu.VMEM((WB + 16,), jnp.float32),
        pltpu.VMEM_SHARED((RB + 8, D), jnp.float32),
        pltpu.SemaphoreType.DMA,
    ],
)
def _sc_encode(fnode_h, fedge_h, nidx_h, eidx_h, rows_h, cols_h, vals_h,
               vembed_h, posrep_h, bounds_h, out_h,
               idxstage, idx2, gbuf, rowstage, rowsbuf,
               bounds_s, vals_s, acc, sem):
    c = lax.axis_index("c")
    t = lax.axis_index("s")
    wid = t * NC + c
    lane = lax.iota(jnp.int32, 16)

    def phase_a(idx_hbm, table_hbm, stride, out_base):
        @pl.loop(0, PER_W // WA)
        def _(w):
            base = wid * PER_W + w * WA
            pltpu.sync_copy(idx_hbm.at[pl.ds(base, WA)], idxstage)
            for q in range(WA // 128):
                for g8 in range(8):
                    g = q * 8 + g8
                    sl = pl.ds(g * 16, 16)
                    gi = base + g * 16 + lane
                    seqv = lax.shift_right_logical(gi, 12)
                    idx2[q, pl.ds(g8 * 16, 16)] = idxstage[sl] + seqv * stride
            cps = [
                pltpu.async_copy(table_hbm.at[idx2.at[q]],
                                 gbuf.at[pl.ds(q * 128, 128)], sem)
                for q in range(WA // 128)
            ]
            for cp in cps:
                cp.wait()
            pltpu.sync_copy(gbuf, out_h.at[pl.ds(out_base + base, WA)])

    phase_a(nidx_h, fnode_h, NTYPES_N, 0)
    phase_a(eidx_h, fedge_h, NTYPES_E, NROWS)

    pltpu.sync_copy(bounds_h, bounds_s.at[pl.ds(0, 16)])

    for bi in range(NBLK // NC):
        b = bi * NC + c
        r0 = b * RB
        sb = bounds_s[pl.ds(b, 16)][0]
        eb = bounds_s[pl.ds(b + 1, 16)][0]

        for ki in range(NCHUNK // NS):
            k = t * (NCHUNK // NS) + ki
            seqq = b * SEQ_PER_BLK + lax.div(k, BATCH // CHUNK)
            pltpu.sync_copy(posrep_h.at[seqq], acc.at[pl.ds(k * CHUNK, CHUNK)])
        plsc.subcore_barrier()

        jlo = lax.div(sb, WB)
        jhi = lax.div(eb + (WB - 1), WB)

        @pl.loop(jlo + t, jhi, step=NS)
        def _(j):
            base = j * WB
            for q in range(WB // 128):
                pltpu.sync_copy(cols_h.at[pl.ds(base + q * 128, 128)],
                                idx2.at[q])
            pltpu.sync_copy(rows_h.at[pl.ds(base, WB)], rowstage)
            pltpu.sync_copy(vals_h.at[pl.ds(base, WB)],
                            vals_s.at[pl.ds(0, WB)])
            for q in range(WB // 128):
                for g8 in range(8):
                    g = q * 8 + g8
                    sl = pl.ds(g * 16, 16)
                    sl8 = pl.ds(g8 * 16, 16)
                    rv = rowstage[sl] - r0
                    oob = (rv < 0) | (rv >= RB)
                    rowsbuf[q, sl8] = jnp.where(oob, RB, rv)
            cps = [
                pltpu.async_copy(vembed_h.at[idx2.at[q]],
                                 gbuf.at[pl.ds(q * 128, 128)], sem)
                for q in range(WB // 128)
            ]
            for cp in cps:
                cp.wait()

            @pl.loop(0, WB)
            def _(i):
                v = vals_s[pl.ds(i, 16)][0]
                for qq in range(D // 16):
                    slq = pl.ds(qq * 16, 16)
                    gbuf[i, slq] = gbuf[i, slq] * v

            for q in range(WB // 128):
                pltpu.sync_copy(gbuf.at[pl.ds(q * 128, 128)],
                                acc.at[rowsbuf.at[q]], add=True)

        plsc.subcore_barrier()
        pltpu.sync_copy(
            acc.at[pl.ds(t * ROWS_PER_TILE, ROWS_PER_TILE)],
            out_h.at[pl.ds(2 * NROWS + r0 + t * ROWS_PER_TILE,
                           ROWS_PER_TILE)])
        plsc.subcore_barrier()


def kernel(node_idx, edge_idx, val_rows, val_cols, val_vals,
           node_table, edge_table, val_embed, pos_node, pos_edge, pos_val):
    fnode, fedge, posrep = _prep(node_table, edge_table,
                                 pos_node, pos_edge, pos_val)
    fnode = fnode.reshape(SEQLEN * NTYPES_N, D)
    fedge = fedge.reshape(SEQLEN * NTYPES_E, D)
    nidx = node_idx.reshape(-1).astype(jnp.int32)
    eidx = edge_idx.reshape(-1).astype(jnp.int32)
    rows = val_rows.astype(jnp.int32)
    cols = val_cols.astype(jnp.int32)
    edges = jnp.arange(0, NROWS + 1, RB, dtype=jnp.int32)
    bounds = jnp.searchsorted(rows, edges).astype(jnp.int32)
    bounds = jnp.concatenate(
        [bounds, jnp.zeros((16 - NBLK - 1,), jnp.int32)])
    out_flat = _sc_encode(fnode, fedge, nidx, eidx, rows, cols,
                          val_vals, val_embed, posrep, bounds)
    return out_flat.reshape(3 * SEQLEN, BATCH, D)
